# baseline (speedup 1.0000x reference)
import functools

import jax
import jax.numpy as jnp
from jax import lax
from jax.experimental import pallas as pl
from jax.experimental.pallas import tpu as pltpu
from jax.experimental.pallas import tpu_sc as plsc

_D = 64
_LANES = 128
_CHUNK = 1024
_K = _CHUNK // _LANES


@functools.partial(jax.jit, static_argnames=("b_total",))
def _gather_sc(idx2d, weight, b_total):
    info = plsc.get_sparse_core_info()
    nw = info.num_cores * info.num_subcores
    b_per_w = b_total // nw
    steps = b_per_w // _CHUNK
    mesh = plsc.VectorSubcoreMesh(core_axis_name="c", subcore_axis_name="s")

    @functools.partial(
        pl.kernel,
        mesh=mesh,
        out_type=jax.ShapeDtypeStruct((b_total, _D), jnp.float32),
        scratch_types=[
            pltpu.VMEM((_K, _LANES), jnp.int32),
            pltpu.VMEM((_CHUNK, _D), jnp.float32),
            pltpu.SemaphoreType.DMA,
        ],
        compiler_params=pltpu.CompilerParams(use_tc_tiling_on_sc=False),
    )
    def k(idx_hbm, w_hbm, out_hbm, idx_v, rows_v, gsem):
        wid = lax.axis_index("s") * info.num_cores + lax.axis_index("c")
        row0 = wid * b_per_w

        def body(j, carry):
            off = row0 + j * _CHUNK
            irow = pl.multiple_of(off // _LANES, 8)
            pltpu.sync_copy(idx_hbm.at[pl.ds(irow, _K)], idx_v)
            cps = [
                pltpu.async_copy(
                    w_hbm.at[idx_v.at[t]],
                    rows_v.at[pl.ds(t * _LANES, _LANES)],
                    gsem,
                )
                for t in range(_K)
            ]
            for cp in cps:
                cp.wait()
            pltpu.sync_copy(rows_v, out_hbm.at[pl.ds(off, _CHUNK)])
            return carry

        lax.fori_loop(0, steps, body, 0)

    return k(idx2d, weight)


def kernel(input_, weight):
    b, t = input_.shape
    b_total = b * t
    idx2d = input_.astype(jnp.int32).reshape(b_total // _LANES, _LANES)
    out = _gather_sc(idx2d, weight, b_total)
    return out.reshape(b, t, _D)


# speedup vs baseline: 1.0105x; 1.0105x over previous
import functools

import jax
import jax.numpy as jnp
from jax import lax
from jax.experimental import pallas as pl
from jax.experimental.pallas import tpu as pltpu
from jax.experimental.pallas import tpu_sc as plsc

_D = 64
_LANES = 128
_CHUNK = 512
_K = _CHUNK // _LANES


@functools.partial(jax.jit, static_argnames=("b_total",))
def _gather_sc(idx2d, weight, b_total):
    info = plsc.get_sparse_core_info()
    nw = info.num_cores * info.num_subcores
    b_per_w = b_total // nw
    irows_per_w = b_per_w // _LANES
    steps = b_per_w // _CHUNK
    half = steps // 2
    mesh = plsc.VectorSubcoreMesh(core_axis_name="c", subcore_axis_name="s")

    @functools.partial(
        pl.kernel,
        mesh=mesh,
        out_type=jax.ShapeDtypeStruct((b_total, _D), jnp.float32),
        scratch_types=[
            pltpu.VMEM((irows_per_w, _LANES), jnp.int32),
            pltpu.VMEM((2, _CHUNK, _D), jnp.float32),
            pltpu.SemaphoreType.DMA,
            pltpu.SemaphoreType.DMA,
            pltpu.SemaphoreType.DMA,
            pltpu.SemaphoreType.DMA,
        ],
        compiler_params=pltpu.CompilerParams(use_tc_tiling_on_sc=False),
    )
    def k(idx_hbm, w_hbm, out_hbm, idx_v, rows_v, g0, g1, w0, w1):
        wid = lax.axis_index("s") * info.num_cores + lax.axis_index("c")
        row0 = wid * b_per_w
        gsems = (g0, g1)
        wsems = (w0, w1)

        pltpu.sync_copy(
            idx_hbm.at[pl.ds(wid * irows_per_w, irows_per_w)], idx_v)

        def desc_g(j, slot, t):
            return pltpu.make_async_copy(
                w_hbm.at[idx_v.at[j * _K + t]],
                rows_v.at[slot, pl.ds(t * _LANES, _LANES)],
                gsems[slot])

        def desc_w(j, slot):
            return pltpu.make_async_copy(
                rows_v.at[slot], out_hbm.at[pl.ds(row0 + j * _CHUNK, _CHUNK)],
                wsems[slot])

        def fire(j, slot):
            for t in range(_K):
                desc_g(j, slot, t).start()

        def wait_g(j, slot):
            for t in range(_K):
                desc_g(j, slot, t).wait()

        fire(0, 0)

        def body(g, carry):
            for slot in (0, 1):
                j = 2 * g + slot
                other = 1 - slot
                if slot == 0:
                    @pl.when(g > 0)
                    def _():
                        desc_w(j - 1, other).wait()
                    fire(j + 1, other)
                else:
                    desc_w(j - 1, other).wait()

                    @pl.when(g < half - 1)
                    def _():
                        fire(j + 1, other)
                wait_g(j, slot)
                desc_w(j, slot).start()
            return carry

        lax.fori_loop(0, half, body, 0)
        desc_w(steps - 1, 1).wait()

    return k(idx2d, weight)


def kernel(input_, weight):
    b, t = input_.shape
    b_total = b * t
    idx2d = input_.astype(jnp.int32).reshape(b_total // _LANES, _LANES)
    out = _gather_sc(idx2d, weight, b_total)
    return out.reshape(b, t, _D)
